# Initial kernel scaffold; baseline (speedup 1.0000x reference)
import jax
import jax.numpy as jnp
from jax.experimental import pallas as pl


def kernel(x_topo, edge_index, h_sem, W1, b1, W2, b2, mt_w1, mt_b1, mt_g, mt_be, mt_w2, mt_b2, ms_w1, ms_b1, ms_g, ms_be, ms_w2, ms_b2):
    raise NotImplementedError("write your pallas kernel here")


# speedup vs baseline: 21.3539x; 21.3539x over previous
import functools

import jax
import jax.numpy as jnp
from jax import lax
from jax.experimental import pallas as pl
from jax.experimental.pallas import tpu as pltpu
from jax.experimental.pallas import tpu_sc as plsc

N = 10000
E = 320000
TOPO_IN = 128
HID = 256
Z = 128
HALF = 128

NC = 2
NS = 16
CHUNK = 80
NPAD = 10240

EPT_DEG = E // (NC * NS)
NCH_DEG = EPT_DEG // CHUNK
EPT = E // NS
NCH = EPT // CHUNK
NSB = 5
NCHB = NCH // NSB
RPAD_PT = NPAD // NS

_MESH = plsc.VectorSubcoreMesh(core_axis_name="c", subcore_axis_name="s")


def _zero_f32(ref, nwords):
  def body(i, _):
    ref[pl.ds(i * 16, 16)] = jnp.zeros((16,), jnp.float32)
    return 0
  lax.fori_loop(0, nwords // 16, body, 0, unroll=4)


def _zero_f32_2d(ref, nrows, ncols):
  def body(i, _):
    for k in range(ncols // 16):
      ref[i, pl.ds(k * 16, 16)] = jnp.zeros((16,), jnp.float32)
    return 0
  lax.fori_loop(0, nrows, body, 0)


def _deg_body(dst3_hbm, deg_out, dstb, ones_v, zb, deg_sp):
  c = lax.axis_index("c")
  s = lax.axis_index("s")
  wid = c * NS + s
  pltpu.sync_copy(dst3_hbm.at[wid], dstb)
  def fill(i, _):
    ones_v[pl.ds(i * 16, 16)] = jnp.ones((16,), jnp.float32)
    return 0
  lax.fori_loop(0, CHUNK // 16, fill, 0)
  _zero_f32(zb, NPAD // NS)
  pltpu.sync_copy(zb, deg_sp.at[pl.ds(s * (NPAD // NS), NPAD // NS)])
  plsc.subcore_barrier()
  def body(j, _):
    pltpu.sync_copy(ones_v, deg_sp.at[dstb.at[j]], add=True)
    return 0
  lax.fori_loop(0, NCH_DEG, body, 0)
  plsc.subcore_barrier()
  pltpu.sync_copy(deg_sp.at[pl.ds(s * (NPAD // NS), NPAD // NS)],
                  deg_out.at[pl.ds(c * NPAD + s * (NPAD // NS), NPAD // NS)])


@jax.jit
def _sc_degree(dst3d):
  return pl.kernel(
      _deg_body,
      out_type=jax.ShapeDtypeStruct((NC * NPAD,), jnp.float32),
      mesh=_MESH,
      scratch_types=[
          pltpu.VMEM((NCH_DEG, CHUNK), jnp.int32),
          pltpu.VMEM((CHUNK,), jnp.float32),
          pltpu.VMEM((NPAD // NS,), jnp.float32),
          pltpu.VMEM_SHARED((NPAD,), jnp.float32),
      ],
  )(dst3d)


NB = 3


def _spmv_body(g_hbm, src4_hbm, dst4_hbm, out_hbm, srcb, dstb, rows, acc_sp,
               sg0, sg1, sg2, ss0, ss1, ss2):
  sg = [sg0, sg1, sg2]
  ss = [ss0, ss1, ss2]
  c = lax.axis_index("c")
  s = lax.axis_index("s")
  cN = c * N
  _zero_f32_2d(rows.at[0], CHUNK, HALF)
  for q in range(RPAD_PT // CHUNK):
    pltpu.sync_copy(rows.at[0],
                    acc_sp.at[pl.ds(s * RPAD_PT + q * CHUNK, CHUNK)])
  plsc.subcore_barrier()

  def gstart(j, b):
    pltpu.async_copy(g_hbm.at[srcb.at[j]], rows.at[b], sg[b])

  def gwait(j, b):
    pltpu.make_async_copy(g_hbm.at[srcb.at[j]], rows.at[b], sg[b]).wait()

  def sstart(j, b):
    pltpu.async_copy(rows.at[b], acc_sp.at[dstb.at[j]], ss[b], add=True)

  def swait(j, b):
    pltpu.make_async_copy(rows.at[b], acc_sp.at[dstb.at[j]], ss[b]).wait()

  def super_block(sb, _):
    pltpu.sync_copy(src4_hbm.at[s, sb], srcb)
    pltpu.sync_copy(dst4_hbm.at[s, sb], dstb)
    def bias(j, _):
      for k in range(CHUNK // 16):
        v = srcb[j, pl.ds(k * 16, 16)]
        srcb[j, pl.ds(k * 16, 16)] = v + cN
      return 0
    lax.fori_loop(0, NCHB, bias, 0)
    for b in range(NB):
      gstart(b, b)
    def group(jg, _):
      j = NB * jg
      for b in range(NB):
        gwait(j + b, b)
        sstart(j + b, b)
      for b in range(NB):
        swait(j + b, b)
        gstart(j + b + NB, b)
      return 0
    lax.fori_loop(0, NCHB // NB - 1, group, 0)
    jt = NB * (NCHB // NB - 1)
    for b in range(NB):
      gwait(jt + b, b)
      sstart(jt + b, b)
    for b in range(NCHB - jt - NB):
      swait(jt + b, b)
      gstart(jt + NB + b, b)
    for b in range(NCHB - jt - NB):
      gwait(jt + NB + b, b)
      sstart(jt + NB + b, b)
    for b in range(NCHB - jt - NB, NB):
      swait(jt + b, b)
    for b in range(NCHB - jt - NB):
      swait(jt + NB + b, b)
    return 0

  lax.fori_loop(0, NSB, super_block, 0)
  plsc.subcore_barrier()
  for q in range(RPAD_PT // CHUNK):
    pltpu.sync_copy(acc_sp.at[pl.ds(s * RPAD_PT + q * CHUNK, CHUNK)],
                    out_hbm.at[pl.ds(c * NPAD + s * RPAD_PT + q * CHUNK, CHUNK)])


@jax.jit
def _sc_spmv(g2n, src3d, dst3d):
  return pl.kernel(
      _spmv_body,
      out_type=jax.ShapeDtypeStruct((NC * NPAD, HALF), jnp.float32),
      mesh=_MESH,
      scratch_types=[
          pltpu.VMEM((NCHB, CHUNK), jnp.int32),
          pltpu.VMEM((NCHB, CHUNK), jnp.int32),
          pltpu.VMEM((NB, CHUNK, HALF), jnp.float32),
          pltpu.VMEM_SHARED((NPAD, HALF), jnp.float32),
      ] + [pltpu.SemaphoreType.DMA] * (2 * NB),
  )(g2n, src3d, dst3d)


RB = 2000
NRB = N // RB


def _tc_g1_body(x_ref, w_ref, dinv_ref, out_ref):
  t = jnp.dot(x_ref[...], w_ref[...], preferred_element_type=jnp.float32)
  out_ref[0] = t[:, :HALF] * dinv_ref[...]
  out_ref[1] = t[:, HALF:] * dinv_ref[...]


@jax.jit
def _tc_g1(x, w1, dinv):
  return pl.pallas_call(
      _tc_g1_body,
      grid=(NRB,),
      in_specs=[
          pl.BlockSpec((RB, TOPO_IN), lambda i: (i, 0)),
          pl.BlockSpec((TOPO_IN, HID), lambda i: (0, 0)),
          pl.BlockSpec((RB, 1), lambda i: (i, 0)),
      ],
      out_specs=pl.BlockSpec((NC, RB, HALF), lambda i: (0, i, 0)),
      out_shape=jax.ShapeDtypeStruct((NC, N, HALF), jnp.float32),
  )(x, w1, dinv)


def _tc_g2_body(acc_ref, g_ref, dinv_ref, b_ref, w_ref, out_ref):
  conv = jnp.concatenate(
      [acc_ref[0] + g_ref[0], acc_ref[1] + g_ref[1]], axis=1)
  h1 = jnp.maximum(dinv_ref[...] * conv + b_ref[...], 0.0)
  t = jnp.dot(h1, w_ref[...], preferred_element_type=jnp.float32)
  out_ref[0] = t[:, :HALF] * dinv_ref[...]
  out_ref[1] = t[:, HALF:] * dinv_ref[...]


@jax.jit
def _tc_g2(acc1, g1, dinv, b1, w2):
  return pl.pallas_call(
      _tc_g2_body,
      grid=(NRB,),
      in_specs=[
          pl.BlockSpec((NC, RB, HALF), lambda i: (0, i, 0)),
          pl.BlockSpec((NC, RB, HALF), lambda i: (0, i, 0)),
          pl.BlockSpec((RB, 1), lambda i: (i, 0)),
          pl.BlockSpec((HID,), lambda i: (0,)),
          pl.BlockSpec((HID, HID), lambda i: (0, 0)),
      ],
      out_specs=pl.BlockSpec((NC, RB, HALF), lambda i: (0, i, 0)),
      out_shape=jax.ShapeDtypeStruct((NC, N, HALF), jnp.float32),
  )(acc1, g1, dinv, b1, w2)


def _tc_topo_final_body(acc_ref, g_ref, dinv_ref, b2_ref, w_ref, bb_ref,
                        s_ref, st_s_ref,
                        tg_ref, tb_ref, tw_ref, tb2_ref,
                        sg_ref, sb_ref, sw_ref, sb2_ref,
                        zt_ref, zs_ref, t_scr, st_scr):
  p = pl.program_id(0)
  i = pl.program_id(1)

  @pl.when(p == 0)
  def _():
    conv = jnp.concatenate(
        [acc_ref[0] + g_ref[0], acc_ref[1] + g_ref[1]], axis=1)
    ht = dinv_ref[...] * conv + b2_ref[...]
    t = jnp.dot(ht, w_ref[...],
                preferred_element_type=jnp.float32) + bb_ref[...]
    t_scr[pl.ds(i * RB, RB), :] = t
    part = jnp.concatenate([jnp.sum(t, axis=0, keepdims=True),
                            jnp.sum(t * t, axis=0, keepdims=True)], axis=0)

    @pl.when(i == 0)
    def _():
      st_scr[...] = part

    @pl.when(i > 0)
    def _():
      st_scr[...] += part

  @pl.when(p == 1)
  def _():
    zt_ref[...] = _bn_head(t_scr[pl.ds(i * RB, RB), :], st_scr[...],
                           tg_ref[...], tb_ref[...], tw_ref[...], tb2_ref[...])
    zs_ref[...] = _bn_head(s_ref[...], st_s_ref[...],
                           sg_ref[...], sb_ref[...], sw_ref[...], sb2_ref[...])


@jax.jit
def _tc_topo_final(acc2, g2, dinv, b2, mw1, mb1, s_pre, st_s,
                   mt_g, mt_be, mt_w2, mt_b2, ms_g, ms_be, ms_w2, ms_b2):
  vec = lambda d: pl.BlockSpec((d,), lambda p, i: (0,))
  return pl.pallas_call(
      _tc_topo_final_body,
      grid=(2, NRB),
      in_specs=[
          pl.BlockSpec((NC, RB, HALF),
                       lambda p, i: (0, jnp.where(p == 0, i, NRB - 1), 0)),
          pl.BlockSpec((NC, RB, HALF),
                       lambda p, i: (0, jnp.where(p == 0, i, NRB - 1), 0)),
          pl.BlockSpec((RB, 1), lambda p, i: (i, 0)),
          pl.BlockSpec((HID,), lambda p, i: (0,)),
          pl.BlockSpec((HID, HID), lambda p, i: (0, 0)),
          pl.BlockSpec((HID,), lambda p, i: (0,)),
          pl.BlockSpec((RB, HID),
                       lambda p, i: (jnp.where(p == 1, i, 0), 0)),
          pl.BlockSpec((2, HID), lambda p, i: (0, 0)),
          vec(HID), vec(HID),
          pl.BlockSpec((HID, Z), lambda p, i: (0, 0)), vec(Z),
          vec(HID), vec(HID),
          pl.BlockSpec((HID, Z), lambda p, i: (0, 0)), vec(Z),
      ],
      out_specs=[
          pl.BlockSpec((RB, Z), lambda p, i: (i, 0)),
          pl.BlockSpec((RB, Z), lambda p, i: (i, 0)),
      ],
      out_shape=[
          jax.ShapeDtypeStruct((N, Z), jnp.float32),
          jax.ShapeDtypeStruct((N, Z), jnp.float32),
      ],
      scratch_shapes=[
          pltpu.VMEM((N, HID), jnp.float32),
          pltpu.VMEM((2, HID), jnp.float32),
      ],
  )(acc2, g2, dinv, b2, mw1, mb1, s_pre, st_s,
    mt_g, mt_be, mt_w2, mt_b2, ms_g, ms_be, ms_w2, ms_b2)


def _tc_sem_pre_body(x_ref, w_ref, b_ref, t_ref, st_ref):
  i = pl.program_id(0)
  t = jnp.dot(x_ref[...], w_ref[...],
              preferred_element_type=jnp.float32) + b_ref[...]
  t_ref[...] = t
  part = jnp.concatenate([jnp.sum(t, axis=0, keepdims=True),
                          jnp.sum(t * t, axis=0, keepdims=True)], axis=0)

  @pl.when(i == 0)
  def _():
    st_ref[...] = part

  @pl.when(i > 0)
  def _():
    st_ref[...] += part


@jax.jit
def _tc_sem_pre(h_sem, mw1, mb1):
  return pl.pallas_call(
      _tc_sem_pre_body,
      grid=(NRB,),
      in_specs=[
          pl.BlockSpec((RB, HID), lambda i: (i, 0)),
          pl.BlockSpec((HID, HID), lambda i: (0, 0)),
          pl.BlockSpec((HID,), lambda i: (0,)),
      ],
      out_specs=[
          pl.BlockSpec((RB, HID), lambda i: (i, 0)),
          pl.BlockSpec((2, HID), lambda i: (0, 0)),
      ],
      out_shape=[
          jax.ShapeDtypeStruct((N, HID), jnp.float32),
          jax.ShapeDtypeStruct((2, HID), jnp.float32),
      ],
  )(h_sem, mw1, mb1)


def _bn_head(t, st, gamma, beta, w2, b2):
  mean = st[0] / N
  var = st[1] / N - mean * mean
  xh = gamma * (t - mean) / jnp.sqrt(var + 1e-5) + beta
  xh = jnp.maximum(xh, 0.0)
  y = jnp.dot(xh, w2, preferred_element_type=jnp.float32) + b2
  nrm = jnp.sqrt(jnp.sum(y * y, axis=1, keepdims=True))
  return y / jnp.maximum(nrm, 1e-12)


def kernel(x_topo, edge_index, h_sem, W1, b1, W2, b2,
           mt_w1, mt_b1, mt_g, mt_be, mt_w2, mt_b2,
           ms_w1, ms_b1, ms_g, ms_be, ms_w2, ms_b2):
  src3d = edge_index[0].reshape(NS, NSB, NCHB, CHUNK)
  dst3d = edge_index[1].reshape(NS, NSB, NCHB, CHUNK)
  dst3d_deg = edge_index[1].reshape(NC * NS, NCH_DEG, CHUNK)

  deg_parts = _sc_degree(dst3d_deg)
  deg = deg_parts[:N] + deg_parts[NPAD:NPAD + N] + 1.0
  dinv = lax.rsqrt(jnp.maximum(deg, 1.0))[:, None]

  s_pre, st_s = _tc_sem_pre(h_sem, ms_w1, ms_b1)

  g1 = _tc_g1(x_topo, W1, dinv)
  acc1 = _sc_spmv(g1.reshape(2 * N, HALF), src3d, dst3d)
  g2 = _tc_g2(acc1.reshape(NC, NPAD, HALF), g1,
              dinv, b1, W2)
  acc2 = _sc_spmv(g2.reshape(2 * N, HALF), src3d, dst3d)
  z_topo, z_sem = _tc_topo_final(acc2.reshape(NC, NPAD, HALF), g2, dinv, b2,
                                 mt_w1, mt_b1, s_pre, st_s,
                                 mt_g, mt_be, mt_w2, mt_b2,
                                 ms_g, ms_be, ms_w2, ms_b2)
  return (z_topo, z_sem)
